# Initial kernel scaffold; baseline (speedup 1.0000x reference)
import jax
import jax.numpy as jnp
from jax.experimental import pallas as pl


def kernel(x, conv1_w, conv1_b, conv2_w, conv2_b, fc1_w, fc1_b, fc2_w, fc2_b):
    raise NotImplementedError("write your pallas kernel here")


# speedup vs baseline: 1.3892x; 1.3892x over previous
import jax
import jax.numpy as jnp
from jax.experimental import pallas as pl
from jax.experimental.pallas import tpu as pltpu

_TB = 512
_H = 28
_KH = 5


def _tpz(w, w_in, p):
    cout, cin, kh, kw = w.shape
    owh = (w_in - kw + 1) // 2
    c = jnp.arange(w_in)[:, None]
    k = c - (2 * jnp.arange(owh)[None, :] + p)
    wt = jnp.transpose(w, (2, 3, 1, 0)).astype(jnp.float32)
    g = jnp.take(wt, jnp.clip(k, 0, kw - 1), axis=1)
    g = jnp.where(((k >= 0) & (k < kw))[None, :, :, None, None], g, 0.0)
    g = jnp.transpose(g, (0, 1, 3, 2, 4))
    return g.reshape(kh, w_in * cin, owh * cout)


def _conv1_groups(w):
    t = jnp.zeros((224, 1024), jnp.float32)
    for pty in range(2):
        flat = _tpz(w, _H, pty).reshape(_KH * _H, 120)
        for j in range(4):
            t = jax.lax.dynamic_update_slice(t, flat, (28 * j, 512 * pty + 128 * j))
    return t.astype(jnp.bfloat16)


def _conv2_mat(w):
    parts = []
    for pty in range(2):
        tp = _tpz(w, 12, pty)
        parts.append(jnp.pad(tp, ((0, 0), (0, 8), (0, 48))))
    return jnp.concatenate(parts, axis=2).reshape(640, 256).astype(jnp.bfloat16)


def _fc1_mat(w1):
    t = w1.astype(jnp.float32).reshape(50, 20, 4, 4)
    t = jnp.transpose(t, (2, 3, 1, 0)).reshape(4, 80, 50)
    t = jnp.pad(t, ((0, 0), (0, 48), (0, 0)))
    return t.reshape(512, 50).astype(jnp.bfloat16)


def _lenet_body(x_ref, t1_ref, t2_ref, w1_ref, w2_ref,
                bc1_ref, bc2_ref, b1_ref, b2_ref, o_ref):
    xb = x_ref[...].astype(jnp.bfloat16)
    t1 = t1_ref[...]
    rows = []
    for g in range(6):
        xg = xb[:, 112 * g:112 * g + 224]
        y = jnp.dot(xg, t1, preferred_element_type=jnp.float32)
        y = jnp.maximum(y[:, :512], y[:, 512:])
        for r in (jnp.maximum(y[:, :128], y[:, 128:256]),
                  jnp.maximum(y[:, 256:384], y[:, 384:512])):
            rows.append(jnp.maximum(r + bc1_ref[...], 0.0).astype(jnp.bfloat16))
    a2 = jnp.concatenate(rows, axis=1)

    t2 = t2_ref[...]
    prow = []
    for j in range(8):
        z = jnp.dot(a2[:, 128 * j:128 * j + 640], t2,
                    preferred_element_type=jnp.float32)
        prow.append(jnp.maximum(z[:, :128], z[:, 128:]))
    arow = [jnp.maximum(jnp.maximum(prow[2 * j], prow[2 * j + 1]) + bc2_ref[...],
                        0.0).astype(jnp.bfloat16)
            for j in range(4)]

    h = jnp.dot(jnp.concatenate(arow, axis=1), w1_ref[...],
                preferred_element_type=jnp.float32)
    h = jnp.maximum(h + b1_ref[...], 0.0).astype(jnp.bfloat16)
    z = jnp.dot(h, w2_ref[...], preferred_element_type=jnp.float32) + b2_ref[...]
    s = z - jnp.max(z, axis=1, keepdims=True)
    o_ref[...] = s - jnp.log(jnp.sum(jnp.exp(s), axis=1, keepdims=True))


def kernel(x, conv1_w, conv1_b, conv2_w, conv2_b, fc1_w, fc1_b, fc2_w, fc2_b):
    n = x.shape[0]
    xf = x.reshape(n, _H * _H).astype(jnp.float32)
    tb = _TB if n >= _TB else max(8, -(-n // 8) * 8)
    n_pad = -(-n // tb) * tb
    if n_pad != n:
        xf = jnp.pad(xf, ((0, n_pad - n), (0, 0)))

    t1 = _conv1_groups(conv1_w)
    t2 = _conv2_mat(conv2_w)
    w1 = _fc1_mat(fc1_w)
    w2 = fc2_w.T.astype(jnp.bfloat16)
    bc1 = jnp.pad(jnp.tile(conv1_b.astype(jnp.float32), 12), (0, 8)).reshape(1, 128)
    bc2 = jnp.pad(jnp.tile(conv2_b.astype(jnp.float32), 4), (0, 48)).reshape(1, 128)
    b1 = fc1_b.astype(jnp.float32).reshape(1, 50)
    b2 = fc2_b.astype(jnp.float32).reshape(1, 10)

    grid = (n_pad // tb,)
    flops = 2 * n_pad * (6 * 224 * 1024 + 8 * 640 * 256 + 512 * 50 + 50 * 10)
    bytes_accessed = 4 * (n_pad * 784 + n_pad * 10) + 2 * (224 * 1024 + 640 * 256)

    out = pl.pallas_call(
        _lenet_body,
        out_shape=jax.ShapeDtypeStruct((n_pad, 10), jnp.float32),
        grid=grid,
        in_specs=[
            pl.BlockSpec((tb, _H * _H), lambda i: (i, 0)),
            pl.BlockSpec((224, 1024), lambda i: (0, 0)),
            pl.BlockSpec((640, 256), lambda i: (0, 0)),
            pl.BlockSpec((512, 50), lambda i: (0, 0)),
            pl.BlockSpec((50, 10), lambda i: (0, 0)),
            pl.BlockSpec((1, 128), lambda i: (0, 0)),
            pl.BlockSpec((1, 128), lambda i: (0, 0)),
            pl.BlockSpec((1, 50), lambda i: (0, 0)),
            pl.BlockSpec((1, 10), lambda i: (0, 0)),
        ],
        out_specs=pl.BlockSpec((tb, 10), lambda i: (i, 0)),
        compiler_params=pltpu.CompilerParams(dimension_semantics=("parallel",)),
        cost_estimate=pl.CostEstimate(flops=flops, transcendentals=n_pad * 11,
                                      bytes_accessed=bytes_accessed),
    )(xf, t1, t2, w1, w2, bc1, bc2, b1, b2)
    return out[:n]


# speedup vs baseline: 1.4275x; 1.0275x over previous
import jax
import jax.numpy as jnp
from jax.experimental import pallas as pl
from jax.experimental.pallas import tpu as pltpu

_TB = 1024
_H = 28
_KH = 5


def _tpz(w, w_in, p):
    cout, cin, kh, kw = w.shape
    owh = (w_in - kw + 1) // 2
    c = jnp.arange(w_in)[:, None]
    k = c - (2 * jnp.arange(owh)[None, :] + p)
    wt = jnp.transpose(w, (2, 3, 1, 0)).astype(jnp.float32)
    g = jnp.take(wt, jnp.clip(k, 0, kw - 1), axis=1)
    g = jnp.where(((k >= 0) & (k < kw))[None, :, :, None, None], g, 0.0)
    g = jnp.transpose(g, (0, 1, 3, 2, 4))
    return g.reshape(kh, w_in * cin, owh * cout)


def _conv1_groups(w):
    t = jnp.zeros((224, 1024), jnp.float32)
    for pty in range(2):
        flat = _tpz(w, _H, pty).reshape(_KH * _H, 120)
        for j in range(4):
            t = jax.lax.dynamic_update_slice(t, flat, (28 * j, 512 * pty + 128 * j))
    return t.astype(jnp.bfloat16)


def _conv2_mat(w):
    parts = []
    for pty in range(2):
        tp = _tpz(w, 12, pty)
        parts.append(jnp.pad(tp, ((0, 0), (0, 8), (0, 48))))
    return jnp.concatenate(parts, axis=2).reshape(640, 256).astype(jnp.bfloat16)


def _fc1_mat(w1):
    t = w1.astype(jnp.float32).reshape(50, 20, 4, 4)
    t = jnp.transpose(t, (2, 3, 1, 0)).reshape(4, 80, 50)
    t = jnp.pad(t, ((0, 0), (0, 48), (0, 0)))
    return t.reshape(512, 50).astype(jnp.bfloat16)


def _lenet_body(x_ref, t1_ref, t2_ref, w1_ref, w2_ref,
                bc1_ref, bc2_ref, b1_ref, b2_ref, o_ref):
    xb = x_ref[...].astype(jnp.bfloat16)
    t1 = t1_ref[...]
    rows = []
    for g in range(6):
        xg = xb[:, 112 * g:112 * g + 224]
        y = jnp.dot(xg, t1, preferred_element_type=jnp.float32)
        y = jnp.maximum(y[:, :512], y[:, 512:])
        for r in (jnp.maximum(y[:, :128], y[:, 128:256]),
                  jnp.maximum(y[:, 256:384], y[:, 384:512])):
            rows.append(jnp.maximum(r + bc1_ref[...], 0.0).astype(jnp.bfloat16))
    a2 = jnp.concatenate(rows, axis=1)

    t2 = t2_ref[...]
    prow = []
    for j in range(8):
        z = jnp.dot(a2[:, 128 * j:128 * j + 640], t2,
                    preferred_element_type=jnp.float32)
        prow.append(jnp.maximum(z[:, :128], z[:, 128:]))
    arow = [jnp.maximum(jnp.maximum(prow[2 * j], prow[2 * j + 1]) + bc2_ref[...],
                        0.0).astype(jnp.bfloat16)
            for j in range(4)]

    h = jnp.dot(jnp.concatenate(arow, axis=1), w1_ref[...],
                preferred_element_type=jnp.float32)
    h = jnp.maximum(h + b1_ref[...], 0.0).astype(jnp.bfloat16)
    z = jnp.dot(h, w2_ref[...], preferred_element_type=jnp.float32) + b2_ref[...]
    s = z - jnp.max(z, axis=1, keepdims=True)
    o_ref[...] = s - jnp.log(jnp.sum(jnp.exp(s), axis=1, keepdims=True))


def kernel(x, conv1_w, conv1_b, conv2_w, conv2_b, fc1_w, fc1_b, fc2_w, fc2_b):
    n = x.shape[0]
    xf = x.reshape(n, _H * _H).astype(jnp.float32)
    tb = _TB if n >= _TB else max(8, -(-n // 8) * 8)
    n_pad = -(-n // tb) * tb
    if n_pad != n:
        xf = jnp.pad(xf, ((0, n_pad - n), (0, 0)))

    t1 = _conv1_groups(conv1_w)
    t2 = _conv2_mat(conv2_w)
    w1 = _fc1_mat(fc1_w)
    w2 = fc2_w.T.astype(jnp.bfloat16)
    bc1 = jnp.pad(jnp.tile(conv1_b.astype(jnp.float32), 12), (0, 8)).reshape(1, 128)
    bc2 = jnp.pad(jnp.tile(conv2_b.astype(jnp.float32), 4), (0, 48)).reshape(1, 128)
    b1 = fc1_b.astype(jnp.float32).reshape(1, 50)
    b2 = fc2_b.astype(jnp.float32).reshape(1, 10)

    grid = (n_pad // tb,)
    flops = 2 * n_pad * (6 * 224 * 1024 + 8 * 640 * 256 + 512 * 50 + 50 * 10)
    bytes_accessed = 4 * (n_pad * 784 + n_pad * 10) + 2 * (224 * 1024 + 640 * 256)

    out = pl.pallas_call(
        _lenet_body,
        out_shape=jax.ShapeDtypeStruct((n_pad, 10), jnp.float32),
        grid=grid,
        in_specs=[
            pl.BlockSpec((tb, _H * _H), lambda i: (i, 0)),
            pl.BlockSpec((224, 1024), lambda i: (0, 0)),
            pl.BlockSpec((640, 256), lambda i: (0, 0)),
            pl.BlockSpec((512, 50), lambda i: (0, 0)),
            pl.BlockSpec((50, 10), lambda i: (0, 0)),
            pl.BlockSpec((1, 128), lambda i: (0, 0)),
            pl.BlockSpec((1, 128), lambda i: (0, 0)),
            pl.BlockSpec((1, 50), lambda i: (0, 0)),
            pl.BlockSpec((1, 10), lambda i: (0, 0)),
        ],
        out_specs=pl.BlockSpec((tb, 10), lambda i: (i, 0)),
        compiler_params=pltpu.CompilerParams(dimension_semantics=("parallel",)),
        cost_estimate=pl.CostEstimate(flops=flops, transcendentals=n_pad * 11,
                                      bytes_accessed=bytes_accessed),
    )(xf, t1, t2, w1, w2, bc1, bc2, b1, b2)
    return out[:n]


# speedup vs baseline: 2.9760x; 2.0848x over previous
import jax
import jax.numpy as jnp
from jax.experimental import pallas as pl
from jax.experimental.pallas import tpu as pltpu

_NB = 2048
_H = 28
_KH = 5


def _tpz(w, w_in, p):
    cout, cin, kh, kw = w.shape
    owh = (w_in - kw + 1) // 2
    c = jnp.arange(w_in)[:, None]
    k = c - (2 * jnp.arange(owh)[None, :] + p)
    wt = jnp.transpose(w, (2, 3, 1, 0)).astype(jnp.float32)
    g = jnp.take(wt, jnp.clip(k, 0, kw - 1), axis=1)
    return jnp.where(((k >= 0) & (k < kw))[None, :, :, None, None], g, 0.0)


def _conv1_mat(w):
    parts = []
    for pty in range(2):
        tp = _tpz(w, _H, pty).reshape(_KH, _H, 120)
        tp = jnp.pad(jnp.transpose(tp, (2, 0, 1)), ((0, 0), (0, 0), (0, 4)))
        parts.append(jnp.stack([jnp.pad(tp, ((0, 0), (j, 3 - j), (0, 0)))
                                for j in range(4)]))
    return jnp.concatenate(parts).reshape(960, 256).astype(jnp.bfloat16)


def _conv2_mat(w):
    parts = []
    for pty in range(2):
        tp = _tpz(w, 12, pty)
        parts.append(jnp.transpose(tp, (2, 4, 0, 1, 3)).reshape(80, _KH * 120))
    return jnp.concatenate(parts).astype(jnp.bfloat16)


def _fc1_mat(w1):
    t = w1.astype(jnp.float32).reshape(50, 20, 4, 4)
    t = jnp.transpose(t, (0, 2, 3, 1))
    return t.reshape(50, 320).astype(jnp.bfloat16)


def _lenet_body(x_ref, t1_ref, t2_ref, w1_ref, w2_ref,
                bc1_ref, bc2_ref, b1_ref, b2_ref, o_ref):
    xb = x_ref[...]
    t1 = t1_ref[...]
    rows = []
    for g in range(6):
        y = jnp.dot(t1, xb[128 * g:128 * g + 256, :],
                    preferred_element_type=jnp.float32).astype(jnp.bfloat16)
        y = jnp.maximum(y[:480], y[480:])
        for r in (jnp.maximum(y[:120], y[120:240]),
                  jnp.maximum(y[240:360], y[360:480])):
            rows.append(jnp.maximum(r + bc1_ref[...], 0.0).astype(jnp.bfloat16))
    a2 = jnp.concatenate(rows, axis=0)

    t2 = t2_ref[...]
    prow = []
    for j in range(8):
        z = jnp.dot(t2, a2[120 * j:120 * j + 600, :],
                    preferred_element_type=jnp.float32).astype(jnp.bfloat16)
        prow.append(jnp.maximum(z[:80], z[80:]))
    arow = [jnp.maximum(jnp.maximum(prow[2 * j], prow[2 * j + 1]) + bc2_ref[...],
                        0.0).astype(jnp.bfloat16)
            for j in range(4)]

    h = jnp.dot(w1_ref[...], jnp.concatenate(arow, axis=0),
                preferred_element_type=jnp.float32)
    h = jnp.maximum(h + b1_ref[...], 0.0).astype(jnp.bfloat16)
    z = jnp.dot(w2_ref[...], h, preferred_element_type=jnp.float32) + b2_ref[...]
    s = z - jnp.max(z, axis=0, keepdims=True)
    o_ref[...] = s - jnp.log(jnp.sum(jnp.exp(s), axis=0, keepdims=True))


def kernel(x, conv1_w, conv1_b, conv2_w, conv2_b, fc1_w, fc1_b, fc2_w, fc2_b):
    n = x.shape[0]
    xh = jnp.transpose(x, (2, 3, 1, 0)).reshape(_H, _H, n)
    xh = jnp.pad(xh.astype(jnp.bfloat16), ((0, 0), (0, 4), (0, 0)))
    nb = _NB if n >= _NB else max(128, -(-n // 128) * 128)
    n_pad = -(-n // nb) * nb
    if n_pad != n:
        xh = jnp.pad(xh, ((0, 0), (0, 0), (0, n_pad - n)))
    xh = xh.reshape(_H * 32, n_pad)

    t1 = _conv1_mat(conv1_w)
    t2 = _conv2_mat(conv2_w)
    w1 = _fc1_mat(fc1_w)
    w2 = fc2_w.astype(jnp.bfloat16)
    bc1 = jnp.tile(conv1_b.astype(jnp.float32), 12).reshape(120, 1)
    bc2 = jnp.tile(conv2_b.astype(jnp.float32), 4).reshape(80, 1)
    b1 = fc1_b.astype(jnp.float32).reshape(50, 1)
    b2 = fc2_b.astype(jnp.float32).reshape(10, 1)

    grid = (n_pad // nb,)
    flops = 2 * n_pad * (6 * 256 * 960 + 8 * 600 * 160 + 320 * 50 + 50 * 10)
    bytes_accessed = 2 * 896 * n_pad + 4 * n_pad * 10 + 2 * (960 * 256 + 600 * 160)

    out = pl.pallas_call(
        _lenet_body,
        out_shape=jax.ShapeDtypeStruct((10, n_pad), jnp.float32),
        grid=grid,
        in_specs=[
            pl.BlockSpec((_H * 32, nb), lambda i: (0, i)),
            pl.BlockSpec((960, 256), lambda i: (0, 0)),
            pl.BlockSpec((160, 600), lambda i: (0, 0)),
            pl.BlockSpec((50, 320), lambda i: (0, 0)),
            pl.BlockSpec((10, 50), lambda i: (0, 0)),
            pl.BlockSpec((120, 1), lambda i: (0, 0)),
            pl.BlockSpec((80, 1), lambda i: (0, 0)),
            pl.BlockSpec((50, 1), lambda i: (0, 0)),
            pl.BlockSpec((10, 1), lambda i: (0, 0)),
        ],
        out_specs=pl.BlockSpec((10, nb), lambda i: (0, i)),
        compiler_params=pltpu.CompilerParams(dimension_semantics=("parallel",)),
        cost_estimate=pl.CostEstimate(flops=flops, transcendentals=n_pad * 11,
                                      bytes_accessed=bytes_accessed),
    )(xh, t1, t2, w1, w2, bc1, bc2, b1, b2)
    return jnp.transpose(out[:, :n])
